# baseline (speedup 1.0000x reference)
import functools

import jax
import jax.numpy as jnp
from jax import lax
from jax.experimental import pallas as pl
from jax.experimental.pallas import tpu as pltpu
from jax.experimental.pallas import tpu_sc as plsc

NUM_SEG = 10000
SEG_BLOCK = 128
ROW_CHUNK = 1024
SC_CHUNK = 80
NUM_WORKERS = 32


def _sc_gather(c0, c1, c2, leaf_table, path_table):
    n = c0.shape[0]
    d = leaf_table.shape[1]
    per_w = n // NUM_WORKERS
    B = SC_CHUNK
    n_chunks = per_w // B
    assert n_chunks % 2 == 1 and n_chunks >= 3
    mesh = plsc.VectorSubcoreMesh(core_axis_name="c", subcore_axis_name="s")
    row_t = jax.ShapeDtypeStruct((n, d), jnp.float32)
    scr = ([pltpu.VMEM((B,), jnp.int32)] * 6
           + [pltpu.VMEM((B, d), jnp.float32)] * 6
           + [pltpu.SemaphoreType.DMA] * 6)

    @functools.partial(pl.kernel, mesh=mesh, out_type=(row_t, row_t, row_t),
                       scratch_types=scr)
    def gather_kernel(c0_h, c1_h, c2_h, leaf_h, path_h, o0_h, o1_h, o2_h,
                      i00, i01, i02, i10, i11, i12,
                      r00, r01, r02, r10, r11, r12,
                      s00, s01, s02, s10, s11, s12):
        ibufs = ((i00, i01, i02), (i10, i11, i12))
        rbufs = ((r00, r01, r02), (r10, r11, r12))
        sems = ((s00, s01, s02), (s10, s11, s12))
        idxs = (c0_h, c1_h, c2_h)
        tabs = (leaf_h, path_h, leaf_h)
        outs = (o0_h, o1_h, o2_h)
        wid = lax.axis_index("s") * 2 + lax.axis_index("c")
        base = wid * per_w

        def stage(c, slot):
            off = base + c * B
            for t in range(3):
                pltpu.sync_copy(idxs[t].at[pl.ds(off, B)], ibufs[slot][t])
            for t in range(3):
                pltpu.async_copy(tabs[t].at[ibufs[slot][t]], rbufs[slot][t],
                                 sems[slot][t])

        def drain(slot):
            for t in range(3):
                pltpu.make_async_copy(tabs[t].at[ibufs[slot][t]],
                                      rbufs[slot][t], sems[slot][t]).wait()

        def writeback(c, slot):
            off = base + c * B
            for t in range(3):
                pltpu.sync_copy(rbufs[slot][t], outs[t].at[pl.ds(off, B)])

        stage(0, 0)

        def outer(g, carry):
            c = 2 * g
            stage(c + 1, 1)
            drain(0)
            writeback(c, 0)
            stage(c + 2, 0)
            drain(1)
            writeback(c + 1, 1)
            return carry

        lax.fori_loop(0, (n_chunks - 1) // 2, outer, 0)
        drain(0)
        writeback(n_chunks - 1, 0)

    return gather_kernel(c0, c1, c2, leaf_table, path_table)


def _segment_fused(ll, pm, lr, idx2, bounds, w1t, w2t, w3t, a2, w_out,
                   b_out2, num_blocks):
    n, d = ll.shape
    code = w1t.shape[1]
    out_dim = w_out.shape[0]
    seg_pad = num_blocks * SEG_BLOCK
    C = ROW_CHUNK

    def body(bounds_r, ll_r, pm_r, lr_r, i_r, w1_r, w2_r, w3_r, a_r,
             wout_r, bout_r, out_r,
             lbuf, pbuf, rbuf, ibuf, sem_l, sem_p, sem_r, sem_i):
        b = pl.program_id(0)
        r0 = bounds_r[b]
        r1 = bounds_r[b + 1]
        c_lo = r0 // C
        c_hi = lax.div(r1 + C - 1, C)
        seg0 = b * SEG_BLOCK

        def start(c, slot):
            off = c * C
            pltpu.make_async_copy(
                ll_r.at[pl.ds(off, C)], lbuf.at[slot], sem_l.at[slot]).start()
            pltpu.make_async_copy(
                pm_r.at[pl.ds(off, C)], pbuf.at[slot], sem_p.at[slot]).start()
            pltpu.make_async_copy(
                lr_r.at[pl.ds(off, C)], rbuf.at[slot], sem_r.at[slot]).start()
            pltpu.make_async_copy(
                i_r.at[:, pl.ds(off, C)], ibuf.at[slot], sem_i.at[slot]).start()

        def wait(c, slot):
            off = c * C
            pltpu.make_async_copy(
                ll_r.at[pl.ds(off, C)], lbuf.at[slot], sem_l.at[slot]).wait()
            pltpu.make_async_copy(
                pm_r.at[pl.ds(off, C)], pbuf.at[slot], sem_p.at[slot]).wait()
            pltpu.make_async_copy(
                lr_r.at[pl.ds(off, C)], rbuf.at[slot], sem_r.at[slot]).wait()
            pltpu.make_async_copy(
                i_r.at[:, pl.ds(off, C)], ibuf.at[slot], sem_i.at[slot]).wait()

        @pl.when(c_lo < c_hi)
        def _():
            start(c_lo, 0)

        def chunk(c, carry):
            m, dnm, acc = carry
            slot = lax.rem(c - c_lo, 2)

            @pl.when(c + 1 < c_hi)
            def _():
                start(c + 1, 1 - slot)

            wait(c, slot)
            z = jnp.dot(lbuf[slot], w1_r[...],
                        preferred_element_type=jnp.float32)
            z = z + jnp.dot(pbuf[slot], w2_r[...],
                            preferred_element_type=jnp.float32)
            z = z + jnp.dot(rbuf[slot], w3_r[...],
                            preferred_element_type=jnp.float32)
            h = jnp.tanh(z)
            sc = lax.dot_general(a_r[...], h, (((1,), (1,)), ((), ())),
                                 preferred_element_type=jnp.float32)
            rel = ibuf[slot] - seg0
            rows = lax.broadcasted_iota(jnp.int32, (SEG_BLOCK, C), 0)
            onehot = rel == rows
            of = onehot.astype(jnp.float32)
            mc = jnp.max(jnp.where(onehot, sc, -1e30), axis=1, keepdims=True)
            m_new = jnp.maximum(m, mc)
            alpha = jnp.exp(m - m_new)
            rowm = lax.dot_general(m_new, of, (((0,), (0,)), ((), ())),
                                   preferred_element_type=jnp.float32)
            ex1 = jnp.exp(jnp.minimum(sc - rowm, 0.0))
            ex = of * ex1
            dnm = dnm * alpha + jnp.sum(ex, axis=1, keepdims=True)
            acc = acc * alpha + jnp.dot(
                ex, h, preferred_element_type=jnp.float32)
            return m_new, dnm, acc

        m0 = jnp.full((SEG_BLOCK, 1), -1e30, jnp.float32)
        d0 = jnp.zeros((SEG_BLOCK, 1), jnp.float32)
        a0 = jnp.zeros((SEG_BLOCK, code), jnp.float32)
        m, dnm, acc = lax.fori_loop(c_lo, c_hi, chunk, (m0, d0, a0))
        v = jnp.where(dnm > 0, acc / jnp.where(dnm > 0, dnm, 1.0), 0.0)
        out = lax.dot_general(
            v, wout_r[...], (((1,), (1,)), ((), ())),
            preferred_element_type=jnp.float32)
        out_r[...] = out + bout_r[...]

    return pl.pallas_call(
        body,
        grid=(num_blocks,),
        in_specs=[
            pl.BlockSpec(memory_space=pltpu.MemorySpace.SMEM),
            pl.BlockSpec(memory_space=pltpu.MemorySpace.HBM),
            pl.BlockSpec(memory_space=pltpu.MemorySpace.HBM),
            pl.BlockSpec(memory_space=pltpu.MemorySpace.HBM),
            pl.BlockSpec(memory_space=pltpu.MemorySpace.HBM),
            pl.BlockSpec((d, code), lambda b: (0, 0)),
            pl.BlockSpec((d, code), lambda b: (0, 0)),
            pl.BlockSpec((d, code), lambda b: (0, 0)),
            pl.BlockSpec((1, code), lambda b: (0, 0)),
            pl.BlockSpec((out_dim, code), lambda b: (0, 0)),
            pl.BlockSpec((1, out_dim), lambda b: (0, 0)),
        ],
        out_specs=pl.BlockSpec((SEG_BLOCK, out_dim), lambda b: (b, 0)),
        out_shape=jax.ShapeDtypeStruct((seg_pad, out_dim), jnp.float32),
        scratch_shapes=[
            pltpu.VMEM((2, C, d), jnp.float32),
            pltpu.VMEM((2, C, d), jnp.float32),
            pltpu.VMEM((2, C, d), jnp.float32),
            pltpu.VMEM((2, 1, C), jnp.int32),
            pltpu.SemaphoreType.DMA((2,)),
            pltpu.SemaphoreType.DMA((2,)),
            pltpu.SemaphoreType.DMA((2,)),
            pltpu.SemaphoreType.DMA((2,)),
        ],
    )(bounds, ll, pm, lr, idx2, w1t, w2t, w3t, a2, w_out, b_out2)


def kernel(contexts, indices, leaf_table, path_table, W_fc, a, W_out, b_out):
    n = contexts.shape[0]
    d = leaf_table.shape[1]
    num_blocks = (NUM_SEG + SEG_BLOCK - 1) // SEG_BLOCK

    ll, pm, lr = _sc_gather(contexts[:, 0], contexts[:, 1], contexts[:, 2],
                            leaf_table, path_table)

    wt = W_fc.T
    seg_starts = jnp.arange(num_blocks, dtype=jnp.int32) * SEG_BLOCK
    bounds = jnp.concatenate([
        jnp.searchsorted(indices, seg_starts).astype(jnp.int32),
        jnp.array([n], jnp.int32),
    ])
    out_full = _segment_fused(ll, pm, lr, indices[None, :], bounds,
                              wt[:d], wt[d:2 * d], wt[2 * d:], a[None, :],
                              W_out, b_out[None, :], num_blocks)
    return out_full[:NUM_SEG]


# speedup vs baseline: 1.0375x; 1.0375x over previous
import functools

import jax
import jax.numpy as jnp
from jax import lax
from jax.experimental import pallas as pl
from jax.experimental.pallas import tpu as pltpu
from jax.experimental.pallas import tpu_sc as plsc

NUM_SEG = 10000
SEG_BLOCK = 128
ROW_CHUNK = 1024
SC_CHUNK = 80
NUM_WORKERS = 32


def _sc_gather(c0, c1, c2, leaf_table, path_table):
    n = c0.shape[0]
    d = leaf_table.shape[1]
    per_w = n // NUM_WORKERS
    B = SC_CHUNK
    n_chunks = per_w // B
    assert n_chunks % 2 == 1 and n_chunks >= 3
    mesh = plsc.VectorSubcoreMesh(core_axis_name="c", subcore_axis_name="s")
    row_t = jax.ShapeDtypeStruct((n, d), jnp.float32)
    scr = ([pltpu.VMEM((B,), jnp.int32)] * 6
           + [pltpu.VMEM((B, d), jnp.float32)] * 6
           + [pltpu.SemaphoreType.DMA] * 6)

    @functools.partial(pl.kernel, mesh=mesh, out_type=(row_t, row_t, row_t),
                       scratch_types=scr)
    def gather_kernel(c0_h, c1_h, c2_h, leaf_h, path_h, o0_h, o1_h, o2_h,
                      i00, i01, i02, i10, i11, i12,
                      r00, r01, r02, r10, r11, r12,
                      s00, s01, s02, s10, s11, s12):
        ibufs = ((i00, i01, i02), (i10, i11, i12))
        rbufs = ((r00, r01, r02), (r10, r11, r12))
        sems = ((s00, s01, s02), (s10, s11, s12))
        idxs = (c0_h, c1_h, c2_h)
        tabs = (leaf_h, path_h, leaf_h)
        outs = (o0_h, o1_h, o2_h)
        wid = lax.axis_index("s") * 2 + lax.axis_index("c")
        base = wid * per_w

        def stage(c, slot):
            off = base + c * B
            for t in range(3):
                pltpu.sync_copy(idxs[t].at[pl.ds(off, B)], ibufs[slot][t])
            for t in range(3):
                pltpu.async_copy(tabs[t].at[ibufs[slot][t]], rbufs[slot][t],
                                 sems[slot][t])

        def drain(slot):
            for t in range(3):
                pltpu.make_async_copy(tabs[t].at[ibufs[slot][t]],
                                      rbufs[slot][t], sems[slot][t]).wait()

        def writeback(c, slot):
            off = base + c * B
            for t in range(3):
                pltpu.sync_copy(rbufs[slot][t], outs[t].at[pl.ds(off, B)])

        stage(0, 0)

        def outer(g, carry):
            c = 2 * g
            stage(c + 1, 1)
            drain(0)
            writeback(c, 0)
            stage(c + 2, 0)
            drain(1)
            writeback(c + 1, 1)
            return carry

        lax.fori_loop(0, (n_chunks - 1) // 2, outer, 0)
        drain(0)
        writeback(n_chunks - 1, 0)

    return gather_kernel(c0, c1, c2, leaf_table, path_table)


def _segment_fused(ll, pm, lr, idx2, bounds, w1t, w2t, w3t, a2, w_out,
                   b_out2, num_blocks):
    n, d = ll.shape
    code = w1t.shape[1]
    out_dim = w_out.shape[0]
    seg_pad = num_blocks * SEG_BLOCK
    C = ROW_CHUNK

    def body(bounds_r, ll_r, pm_r, lr_r, i_r, w1_r, w2_r, w3_r, a_r,
             wout_r, bout_r, out_r,
             lbuf, pbuf, rbuf, ibuf, sem_l, sem_p, sem_r, sem_i):
        b = pl.program_id(0)
        r0 = bounds_r[b]
        r1 = bounds_r[b + 1]
        c_lo = r0 // C
        c_hi = lax.div(r1 + C - 1, C)
        seg0 = b * SEG_BLOCK

        def start(c, slot):
            off = c * C
            pltpu.make_async_copy(
                ll_r.at[pl.ds(off, C)], lbuf.at[slot], sem_l.at[slot]).start()
            pltpu.make_async_copy(
                pm_r.at[pl.ds(off, C)], pbuf.at[slot], sem_p.at[slot]).start()
            pltpu.make_async_copy(
                lr_r.at[pl.ds(off, C)], rbuf.at[slot], sem_r.at[slot]).start()
            pltpu.make_async_copy(
                i_r.at[:, pl.ds(off, C)], ibuf.at[slot], sem_i.at[slot]).start()

        def wait(c, slot):
            off = c * C
            pltpu.make_async_copy(
                ll_r.at[pl.ds(off, C)], lbuf.at[slot], sem_l.at[slot]).wait()
            pltpu.make_async_copy(
                pm_r.at[pl.ds(off, C)], pbuf.at[slot], sem_p.at[slot]).wait()
            pltpu.make_async_copy(
                lr_r.at[pl.ds(off, C)], rbuf.at[slot], sem_r.at[slot]).wait()
            pltpu.make_async_copy(
                i_r.at[:, pl.ds(off, C)], ibuf.at[slot], sem_i.at[slot]).wait()

        @pl.when(c_lo < c_hi)
        def _():
            start(c_lo, 0)

        def chunk(c, carry):
            m, dnm, acc = carry
            slot = lax.rem(c - c_lo, 2)

            @pl.when(c + 1 < c_hi)
            def _():
                start(c + 1, 1 - slot)

            wait(c, slot)
            z = jnp.dot(lbuf[slot].astype(jnp.bfloat16), w1_r[...],
                        preferred_element_type=jnp.float32)
            z = z + jnp.dot(pbuf[slot].astype(jnp.bfloat16), w2_r[...],
                            preferred_element_type=jnp.float32)
            z = z + jnp.dot(rbuf[slot].astype(jnp.bfloat16), w3_r[...],
                            preferred_element_type=jnp.float32)
            h = jnp.tanh(z)
            sc = lax.dot_general(a_r[...], h, (((1,), (1,)), ((), ())),
                                 preferred_element_type=jnp.float32)
            rel = ibuf[slot] - seg0
            rows = lax.broadcasted_iota(jnp.int32, (SEG_BLOCK, C), 0)
            onehot = rel == rows
            mc = jnp.max(jnp.where(onehot, sc, -1e30), axis=1, keepdims=True)
            m_new = jnp.maximum(m, mc)
            alpha = jnp.exp(m - m_new)
            ex = jnp.exp(jnp.where(onehot, sc - m_new, -1e30))
            dnm = dnm * alpha + jnp.sum(ex, axis=1, keepdims=True)
            acc = acc * alpha + jnp.dot(
                ex, h, preferred_element_type=jnp.float32)
            return m_new, dnm, acc

        m0 = jnp.full((SEG_BLOCK, 1), -1e30, jnp.float32)
        d0 = jnp.zeros((SEG_BLOCK, 1), jnp.float32)
        a0 = jnp.zeros((SEG_BLOCK, code), jnp.float32)
        m, dnm, acc = lax.fori_loop(c_lo, c_hi, chunk, (m0, d0, a0))
        v = jnp.where(dnm > 0, acc / jnp.where(dnm > 0, dnm, 1.0), 0.0)
        out = lax.dot_general(
            v, wout_r[...], (((1,), (1,)), ((), ())),
            preferred_element_type=jnp.float32)
        out_r[...] = out + bout_r[...]

    return pl.pallas_call(
        body,
        grid=(num_blocks,),
        in_specs=[
            pl.BlockSpec(memory_space=pltpu.MemorySpace.SMEM),
            pl.BlockSpec(memory_space=pltpu.MemorySpace.HBM),
            pl.BlockSpec(memory_space=pltpu.MemorySpace.HBM),
            pl.BlockSpec(memory_space=pltpu.MemorySpace.HBM),
            pl.BlockSpec(memory_space=pltpu.MemorySpace.HBM),
            pl.BlockSpec((d, code), lambda b: (0, 0)),
            pl.BlockSpec((d, code), lambda b: (0, 0)),
            pl.BlockSpec((d, code), lambda b: (0, 0)),
            pl.BlockSpec((1, code), lambda b: (0, 0)),
            pl.BlockSpec((out_dim, code), lambda b: (0, 0)),
            pl.BlockSpec((1, out_dim), lambda b: (0, 0)),
        ],
        out_specs=pl.BlockSpec((SEG_BLOCK, out_dim), lambda b: (b, 0)),
        out_shape=jax.ShapeDtypeStruct((seg_pad, out_dim), jnp.float32),
        scratch_shapes=[
            pltpu.VMEM((2, C, d), jnp.float32),
            pltpu.VMEM((2, C, d), jnp.float32),
            pltpu.VMEM((2, C, d), jnp.float32),
            pltpu.VMEM((2, 1, C), jnp.int32),
            pltpu.SemaphoreType.DMA((2,)),
            pltpu.SemaphoreType.DMA((2,)),
            pltpu.SemaphoreType.DMA((2,)),
            pltpu.SemaphoreType.DMA((2,)),
        ],
    )(bounds, ll, pm, lr, idx2, w1t, w2t, w3t, a2, w_out, b_out2)


def kernel(contexts, indices, leaf_table, path_table, W_fc, a, W_out, b_out):
    n = contexts.shape[0]
    d = leaf_table.shape[1]
    num_blocks = (NUM_SEG + SEG_BLOCK - 1) // SEG_BLOCK

    ll, pm, lr = _sc_gather(contexts[:, 0], contexts[:, 1], contexts[:, 2],
                            leaf_table, path_table)

    wt = W_fc.T
    seg_starts = jnp.arange(num_blocks, dtype=jnp.int32) * SEG_BLOCK
    bounds = jnp.concatenate([
        jnp.searchsorted(indices, seg_starts).astype(jnp.int32),
        jnp.array([n], jnp.int32),
    ])
    wb = wt.astype(jnp.bfloat16)
    out_full = _segment_fused(ll, pm, lr, indices[None, :], bounds,
                              wb[:d], wb[d:2 * d], wb[2 * d:], a[None, :],
                              W_out, b_out[None, :], num_blocks)
    return out_full[:NUM_SEG]


# speedup vs baseline: 1.0609x; 1.0226x over previous
import functools

import jax
import jax.numpy as jnp
from jax import lax
from jax.experimental import pallas as pl
from jax.experimental.pallas import tpu as pltpu
from jax.experimental.pallas import tpu_sc as plsc

NUM_SEG = 10000
SEG_BLOCK = 256
ROW_CHUNK = 1024
SC_CHUNK = 80
NUM_WORKERS = 32


def _sc_gather(c0, c1, c2, leaf_table, path_table):
    n = c0.shape[0]
    d = leaf_table.shape[1]
    per_w = n // NUM_WORKERS
    B = SC_CHUNK
    n_chunks = per_w // B
    assert n_chunks % 2 == 1 and n_chunks >= 3
    mesh = plsc.VectorSubcoreMesh(core_axis_name="c", subcore_axis_name="s")
    row_t = jax.ShapeDtypeStruct((n, d), leaf_table.dtype)
    scr = ([pltpu.VMEM((B,), jnp.int32)] * 6
           + [pltpu.VMEM((B, d), leaf_table.dtype)] * 6
           + [pltpu.SemaphoreType.DMA] * 6)

    @functools.partial(pl.kernel, mesh=mesh, out_type=(row_t, row_t, row_t),
                       scratch_types=scr)
    def gather_kernel(c0_h, c1_h, c2_h, leaf_h, path_h, o0_h, o1_h, o2_h,
                      i00, i01, i02, i10, i11, i12,
                      r00, r01, r02, r10, r11, r12,
                      s00, s01, s02, s10, s11, s12):
        ibufs = ((i00, i01, i02), (i10, i11, i12))
        rbufs = ((r00, r01, r02), (r10, r11, r12))
        sems = ((s00, s01, s02), (s10, s11, s12))
        idxs = (c0_h, c1_h, c2_h)
        tabs = (leaf_h, path_h, leaf_h)
        outs = (o0_h, o1_h, o2_h)
        wid = lax.axis_index("s") * 2 + lax.axis_index("c")
        base = wid * per_w

        def stage(c, slot):
            off = base + c * B
            for t in range(3):
                pltpu.sync_copy(idxs[t].at[pl.ds(off, B)], ibufs[slot][t])
            for t in range(3):
                pltpu.async_copy(tabs[t].at[ibufs[slot][t]], rbufs[slot][t],
                                 sems[slot][t])

        def drain(slot):
            for t in range(3):
                pltpu.make_async_copy(tabs[t].at[ibufs[slot][t]],
                                      rbufs[slot][t], sems[slot][t]).wait()

        def writeback(c, slot):
            off = base + c * B
            for t in range(3):
                pltpu.sync_copy(rbufs[slot][t], outs[t].at[pl.ds(off, B)])

        stage(0, 0)

        def outer(g, carry):
            c = 2 * g
            stage(c + 1, 1)
            drain(0)
            writeback(c, 0)
            stage(c + 2, 0)
            drain(1)
            writeback(c + 1, 1)
            return carry

        lax.fori_loop(0, (n_chunks - 1) // 2, outer, 0)
        drain(0)
        writeback(n_chunks - 1, 0)

    return gather_kernel(c0, c1, c2, leaf_table, path_table)


def _segment_fused(ll, pm, lr, idx2, bounds, w1t, w2t, w3t, a2, w_out,
                   b_out2, num_blocks):
    n, d = ll.shape
    code = w1t.shape[1]
    out_dim = w_out.shape[0]
    seg_pad = num_blocks * SEG_BLOCK
    C = ROW_CHUNK

    def body(bounds_r, ll_r, pm_r, lr_r, i_r, w1_r, w2_r, w3_r, a_r,
             wout_r, bout_r, out_r,
             lbuf, pbuf, rbuf, ibuf, sem_l, sem_p, sem_r, sem_i):
        b = pl.program_id(0)
        r0 = bounds_r[b]
        r1 = bounds_r[b + 1]
        c_lo = r0 // C
        c_hi = lax.div(r1 + C - 1, C)
        seg0 = b * SEG_BLOCK

        def start(c, slot):
            off = c * C
            pltpu.make_async_copy(
                ll_r.at[pl.ds(off, C)], lbuf.at[slot], sem_l.at[slot]).start()
            pltpu.make_async_copy(
                pm_r.at[pl.ds(off, C)], pbuf.at[slot], sem_p.at[slot]).start()
            pltpu.make_async_copy(
                lr_r.at[pl.ds(off, C)], rbuf.at[slot], sem_r.at[slot]).start()
            pltpu.make_async_copy(
                i_r.at[:, pl.ds(off, C)], ibuf.at[slot], sem_i.at[slot]).start()

        def wait(c, slot):
            off = c * C
            pltpu.make_async_copy(
                ll_r.at[pl.ds(off, C)], lbuf.at[slot], sem_l.at[slot]).wait()
            pltpu.make_async_copy(
                pm_r.at[pl.ds(off, C)], pbuf.at[slot], sem_p.at[slot]).wait()
            pltpu.make_async_copy(
                lr_r.at[pl.ds(off, C)], rbuf.at[slot], sem_r.at[slot]).wait()
            pltpu.make_async_copy(
                i_r.at[:, pl.ds(off, C)], ibuf.at[slot], sem_i.at[slot]).wait()

        @pl.when(c_lo < c_hi)
        def _():
            start(c_lo, 0)

        def chunk(c, carry):
            m, dnm, acc = carry
            slot = lax.rem(c - c_lo, 2)

            @pl.when(c + 1 < c_hi)
            def _():
                start(c + 1, 1 - slot)

            wait(c, slot)
            z = jnp.dot(lbuf[slot], w1_r[...],
                        preferred_element_type=jnp.float32)
            z = z + jnp.dot(pbuf[slot], w2_r[...],
                            preferred_element_type=jnp.float32)
            z = z + jnp.dot(rbuf[slot], w3_r[...],
                            preferred_element_type=jnp.float32)
            h = jnp.tanh(z)
            sc = lax.dot_general(a_r[...], h, (((1,), (1,)), ((), ())),
                                 preferred_element_type=jnp.float32)
            rel = ibuf[slot] - seg0
            rows = lax.broadcasted_iota(jnp.int32, (SEG_BLOCK, C), 0)
            onehot = rel == rows
            mc = jnp.max(jnp.where(onehot, sc, -1e30), axis=1, keepdims=True)
            m_new = jnp.maximum(m, mc)
            alpha = jnp.exp(m - m_new)
            ex = jnp.exp(jnp.where(onehot, sc - m_new, -1e30))
            dnm = dnm * alpha + jnp.sum(ex, axis=1, keepdims=True)
            acc = acc * alpha + jnp.dot(
                ex, h, preferred_element_type=jnp.float32)
            return m_new, dnm, acc

        m0 = jnp.full((SEG_BLOCK, 1), -1e30, jnp.float32)
        d0 = jnp.zeros((SEG_BLOCK, 1), jnp.float32)
        a0 = jnp.zeros((SEG_BLOCK, code), jnp.float32)
        m, dnm, acc = lax.fori_loop(c_lo, c_hi, chunk, (m0, d0, a0))
        v = jnp.where(dnm > 0, acc / jnp.where(dnm > 0, dnm, 1.0), 0.0)
        out = lax.dot_general(
            v, wout_r[...], (((1,), (1,)), ((), ())),
            preferred_element_type=jnp.float32)
        out_r[...] = out + bout_r[...]

    return pl.pallas_call(
        body,
        grid=(num_blocks,),
        in_specs=[
            pl.BlockSpec(memory_space=pltpu.MemorySpace.SMEM),
            pl.BlockSpec(memory_space=pltpu.MemorySpace.HBM),
            pl.BlockSpec(memory_space=pltpu.MemorySpace.HBM),
            pl.BlockSpec(memory_space=pltpu.MemorySpace.HBM),
            pl.BlockSpec(memory_space=pltpu.MemorySpace.HBM),
            pl.BlockSpec((d, code), lambda b: (0, 0)),
            pl.BlockSpec((d, code), lambda b: (0, 0)),
            pl.BlockSpec((d, code), lambda b: (0, 0)),
            pl.BlockSpec((1, code), lambda b: (0, 0)),
            pl.BlockSpec((out_dim, code), lambda b: (0, 0)),
            pl.BlockSpec((1, out_dim), lambda b: (0, 0)),
        ],
        out_specs=pl.BlockSpec((SEG_BLOCK, out_dim), lambda b: (b, 0)),
        out_shape=jax.ShapeDtypeStruct((seg_pad, out_dim), jnp.float32),
        scratch_shapes=[
            pltpu.VMEM((2, C, d), ll.dtype),
            pltpu.VMEM((2, C, d), ll.dtype),
            pltpu.VMEM((2, C, d), ll.dtype),
            pltpu.VMEM((2, 1, C), jnp.int32),
            pltpu.SemaphoreType.DMA((2,)),
            pltpu.SemaphoreType.DMA((2,)),
            pltpu.SemaphoreType.DMA((2,)),
            pltpu.SemaphoreType.DMA((2,)),
        ],
    )(bounds, ll, pm, lr, idx2, w1t, w2t, w3t, a2, w_out, b_out2)


def kernel(contexts, indices, leaf_table, path_table, W_fc, a, W_out, b_out):
    n = contexts.shape[0]
    d = leaf_table.shape[1]
    num_blocks = (NUM_SEG + SEG_BLOCK - 1) // SEG_BLOCK

    ll, pm, lr = _sc_gather(contexts[:, 0], contexts[:, 1], contexts[:, 2],
                            leaf_table, path_table)

    wt = W_fc.T
    seg_starts = jnp.arange(num_blocks, dtype=jnp.int32) * SEG_BLOCK
    bounds = jnp.concatenate([
        jnp.searchsorted(indices, seg_starts).astype(jnp.int32),
        jnp.array([n], jnp.int32),
    ])
    out_full = _segment_fused(ll, pm, lr, indices[None, :], bounds,
                              wt[:d], wt[d:2 * d], wt[2 * d:], a[None, :],
                              W_out, b_out[None, :], num_blocks)
    return out_full[:NUM_SEG]


# speedup vs baseline: 1.0616x; 1.0006x over previous
import functools

import jax
import jax.numpy as jnp
from jax import lax
from jax.experimental import pallas as pl
from jax.experimental.pallas import tpu as pltpu
from jax.experimental.pallas import tpu_sc as plsc

NUM_SEG = 10000
SEG_BLOCK = 256
ROW_CHUNK = 1024
SC_CHUNK = 80
NUM_WORKERS = 32


def _sc_gather(c0, c1, c2, leaf_table, path_table):
    n = c0.shape[0]
    d = leaf_table.shape[1]
    per_w = n // NUM_WORKERS
    B = SC_CHUNK
    n_chunks = per_w // B
    assert n_chunks % 2 == 1 and n_chunks >= 3
    mesh = plsc.VectorSubcoreMesh(core_axis_name="c", subcore_axis_name="s")
    row_t = jax.ShapeDtypeStruct((n, d), leaf_table.dtype)
    scr = ([pltpu.VMEM((B,), jnp.int32)] * 6
           + [pltpu.VMEM((B, d), leaf_table.dtype)] * 6
           + [pltpu.SemaphoreType.DMA] * 6)

    @functools.partial(pl.kernel, mesh=mesh, out_type=(row_t, row_t, row_t),
                       scratch_types=scr)
    def gather_kernel(c0_h, c1_h, c2_h, leaf_h, path_h, o0_h, o1_h, o2_h,
                      i00, i01, i02, i10, i11, i12,
                      r00, r01, r02, r10, r11, r12,
                      s00, s01, s02, s10, s11, s12):
        ibufs = ((i00, i01, i02), (i10, i11, i12))
        rbufs = ((r00, r01, r02), (r10, r11, r12))
        sems = ((s00, s01, s02), (s10, s11, s12))
        idxs = (c0_h, c1_h, c2_h)
        tabs = (leaf_h, path_h, leaf_h)
        outs = (o0_h, o1_h, o2_h)
        wid = lax.axis_index("s") * 2 + lax.axis_index("c")
        base = wid * per_w

        def stage(c, slot):
            off = base + c * B
            for t in range(3):
                pltpu.sync_copy(idxs[t].at[pl.ds(off, B)], ibufs[slot][t])
            for t in range(3):
                pltpu.async_copy(tabs[t].at[ibufs[slot][t]], rbufs[slot][t],
                                 sems[slot][t])

        def drain(slot):
            for t in range(3):
                pltpu.make_async_copy(tabs[t].at[ibufs[slot][t]],
                                      rbufs[slot][t], sems[slot][t]).wait()

        def writeback(c, slot):
            off = base + c * B
            for t in range(3):
                pltpu.sync_copy(rbufs[slot][t], outs[t].at[pl.ds(off, B)])

        stage(0, 0)

        def outer(g, carry):
            c = 2 * g
            stage(c + 1, 1)
            drain(0)
            writeback(c, 0)
            stage(c + 2, 0)
            drain(1)
            writeback(c + 1, 1)
            return carry

        lax.fori_loop(0, (n_chunks - 1) // 2, outer, 0)
        drain(0)
        writeback(n_chunks - 1, 0)

    return gather_kernel(c0, c1, c2, leaf_table, path_table)


def _segment_fused(ll, pm, lr, idx2, bounds, w1t, w2t, w3t, a2, w_out,
                   b_out2, num_blocks):
    n, d = ll.shape
    code = w1t.shape[1]
    out_dim = w_out.shape[0]
    seg_pad = num_blocks * SEG_BLOCK
    C = ROW_CHUNK

    def body(bounds_r, ll_r, pm_r, lr_r, i_r, w1_r, w2_r, w3_r, a_r,
             wout_r, bout_r, out_r,
             lbuf, pbuf, rbuf, ibuf, sem_l, sem_p, sem_r, sem_i):
        b = pl.program_id(0)
        r0 = bounds_r[b]
        r1 = bounds_r[b + 1]
        c_lo = r0 // C
        c_hi = lax.div(r1 + C - 1, C)
        seg0 = b * SEG_BLOCK

        n_clamp = n - C

        def start(c, slot):
            off = jnp.minimum(c * C, n_clamp)
            pltpu.make_async_copy(
                ll_r.at[pl.ds(off, C)], lbuf.at[slot], sem_l.at[slot]).start()
            pltpu.make_async_copy(
                pm_r.at[pl.ds(off, C)], pbuf.at[slot], sem_p.at[slot]).start()
            pltpu.make_async_copy(
                lr_r.at[pl.ds(off, C)], rbuf.at[slot], sem_r.at[slot]).start()
            pltpu.make_async_copy(
                i_r.at[:, pl.ds(off, C)], ibuf.at[slot], sem_i.at[slot]).start()

        def wait(c, slot):
            off = jnp.minimum(c * C, n_clamp)
            pltpu.make_async_copy(
                ll_r.at[pl.ds(off, C)], lbuf.at[slot], sem_l.at[slot]).wait()
            pltpu.make_async_copy(
                pm_r.at[pl.ds(off, C)], pbuf.at[slot], sem_p.at[slot]).wait()
            pltpu.make_async_copy(
                lr_r.at[pl.ds(off, C)], rbuf.at[slot], sem_r.at[slot]).wait()
            pltpu.make_async_copy(
                i_r.at[:, pl.ds(off, C)], ibuf.at[slot], sem_i.at[slot]).wait()

        @pl.when(c_lo < c_hi)
        def _():
            start(c_lo, 0)

        def chunk(c, carry):
            m, dnm, acc = carry
            slot = lax.rem(c - c_lo, 2)

            @pl.when(c + 1 < c_hi)
            def _():
                start(c + 1, 1 - slot)

            wait(c, slot)
            z = jnp.dot(lbuf[slot], w1_r[...],
                        preferred_element_type=jnp.float32)
            z = z + jnp.dot(pbuf[slot], w2_r[...],
                            preferred_element_type=jnp.float32)
            z = z + jnp.dot(rbuf[slot], w3_r[...],
                            preferred_element_type=jnp.float32)
            h = jnp.tanh(z)
            sc = lax.dot_general(a_r[...], h, (((1,), (1,)), ((), ())),
                                 preferred_element_type=jnp.float32)
            off = jnp.minimum(c * C, n_clamp)
            pos = off + lax.broadcasted_iota(jnp.int32, (1, C), 1)
            rel = jnp.where(pos >= c * C, ibuf[slot] - seg0, -1)
            rows = lax.broadcasted_iota(jnp.int32, (SEG_BLOCK, C), 0)
            onehot = rel == rows
            mc = jnp.max(jnp.where(onehot, sc, -1e30), axis=1, keepdims=True)
            m_new = jnp.maximum(m, mc)
            alpha = jnp.exp(m - m_new)
            ex = jnp.exp(jnp.where(onehot, sc - m_new, -1e30))
            dnm = dnm * alpha + jnp.sum(ex, axis=1, keepdims=True)
            acc = acc * alpha + jnp.dot(
                ex, h, preferred_element_type=jnp.float32)
            return m_new, dnm, acc

        m0 = jnp.full((SEG_BLOCK, 1), -1e30, jnp.float32)
        d0 = jnp.zeros((SEG_BLOCK, 1), jnp.float32)
        a0 = jnp.zeros((SEG_BLOCK, code), jnp.float32)
        m, dnm, acc = lax.fori_loop(c_lo, c_hi, chunk, (m0, d0, a0))
        v = jnp.where(dnm > 0, acc / jnp.where(dnm > 0, dnm, 1.0), 0.0)
        out = lax.dot_general(
            v, wout_r[...], (((1,), (1,)), ((), ())),
            preferred_element_type=jnp.float32)
        out_r[...] = out + bout_r[...]

    return pl.pallas_call(
        body,
        grid=(num_blocks,),
        in_specs=[
            pl.BlockSpec(memory_space=pltpu.MemorySpace.SMEM),
            pl.BlockSpec(memory_space=pltpu.MemorySpace.HBM),
            pl.BlockSpec(memory_space=pltpu.MemorySpace.HBM),
            pl.BlockSpec(memory_space=pltpu.MemorySpace.HBM),
            pl.BlockSpec(memory_space=pltpu.MemorySpace.HBM),
            pl.BlockSpec((d, code), lambda b: (0, 0)),
            pl.BlockSpec((d, code), lambda b: (0, 0)),
            pl.BlockSpec((d, code), lambda b: (0, 0)),
            pl.BlockSpec((1, code), lambda b: (0, 0)),
            pl.BlockSpec((out_dim, code), lambda b: (0, 0)),
            pl.BlockSpec((1, out_dim), lambda b: (0, 0)),
        ],
        out_specs=pl.BlockSpec((SEG_BLOCK, out_dim), lambda b: (b, 0)),
        out_shape=jax.ShapeDtypeStruct((seg_pad, out_dim), jnp.float32),
        scratch_shapes=[
            pltpu.VMEM((2, C, d), ll.dtype),
            pltpu.VMEM((2, C, d), ll.dtype),
            pltpu.VMEM((2, C, d), ll.dtype),
            pltpu.VMEM((2, 1, C), jnp.int32),
            pltpu.SemaphoreType.DMA((2,)),
            pltpu.SemaphoreType.DMA((2,)),
            pltpu.SemaphoreType.DMA((2,)),
            pltpu.SemaphoreType.DMA((2,)),
        ],
    )(bounds, ll, pm, lr, idx2, w1t, w2t, w3t, a2, w_out, b_out2)


def kernel(contexts, indices, leaf_table, path_table, W_fc, a, W_out, b_out):
    n = contexts.shape[0]
    d = leaf_table.shape[1]
    num_blocks = (NUM_SEG + SEG_BLOCK - 1) // SEG_BLOCK

    ll, pm, lr = _sc_gather(contexts[:, 0], contexts[:, 1], contexts[:, 2],
                            leaf_table, path_table)

    wt = W_fc.T
    seg_starts = jnp.arange(num_blocks, dtype=jnp.int32) * SEG_BLOCK
    bounds = jnp.concatenate([
        jnp.searchsorted(indices, seg_starts).astype(jnp.int32),
        jnp.array([n], jnp.int32),
    ])
    out_full = _segment_fused(ll, pm, lr, indices[None, :], bounds,
                              wt[:d], wt[d:2 * d], wt[2 * d:], a[None, :],
                              W_out, b_out[None, :], num_blocks)
    return out_full[:NUM_SEG]


# speedup vs baseline: 1.1714x; 1.1034x over previous
import functools

import jax
import jax.numpy as jnp
from jax import lax
from jax.experimental import pallas as pl
from jax.experimental.pallas import tpu as pltpu
from jax.experimental.pallas import tpu_sc as plsc

NUM_SEG = 10000
SEG_BLOCK = 256
ROW_CHUNK = 1024
SC_CHUNK = 80
NUM_WORKERS = 32
SPLIT_ROWS = 158720


def _sc_gather(c0, c1, c2, leaf_table, path_table):
    n = c0.shape[0]
    d = leaf_table.shape[1]
    per_w = n // NUM_WORKERS
    B = SC_CHUNK
    n_chunks = per_w // B
    assert n_chunks >= 3 and per_w % B == 0 and n % NUM_WORKERS == 0
    mesh = plsc.VectorSubcoreMesh(core_axis_name="c", subcore_axis_name="s")
    row_t = jax.ShapeDtypeStruct((n, d), leaf_table.dtype)
    scr = ([pltpu.VMEM((B,), jnp.int32)] * 6
           + [pltpu.VMEM((B, d), leaf_table.dtype)] * 6
           + [pltpu.SemaphoreType.DMA] * 6)

    @functools.partial(pl.kernel, mesh=mesh, out_type=(row_t, row_t, row_t),
                       scratch_types=scr)
    def gather_kernel(c0_h, c1_h, c2_h, leaf_h, path_h, o0_h, o1_h, o2_h,
                      i00, i01, i02, i10, i11, i12,
                      r00, r01, r02, r10, r11, r12,
                      s00, s01, s02, s10, s11, s12):
        ibufs = ((i00, i01, i02), (i10, i11, i12))
        rbufs = ((r00, r01, r02), (r10, r11, r12))
        sems = ((s00, s01, s02), (s10, s11, s12))
        idxs = (c0_h, c1_h, c2_h)
        tabs = (leaf_h, path_h, leaf_h)
        outs = (o0_h, o1_h, o2_h)
        wid = lax.axis_index("s") * 2 + lax.axis_index("c")
        base = wid * per_w

        def stage(c, slot):
            off = base + c * B
            for t in range(3):
                pltpu.sync_copy(idxs[t].at[pl.ds(off, B)], ibufs[slot][t])
            for t in range(3):
                pltpu.async_copy(tabs[t].at[ibufs[slot][t]], rbufs[slot][t],
                                 sems[slot][t])

        def drain(slot):
            for t in range(3):
                pltpu.make_async_copy(tabs[t].at[ibufs[slot][t]],
                                      rbufs[slot][t], sems[slot][t]).wait()

        def writeback(c, slot):
            off = base + c * B
            for t in range(3):
                pltpu.sync_copy(rbufs[slot][t], outs[t].at[pl.ds(off, B)])

        stage(0, 0)

        def outer(g, carry):
            c = 2 * g
            stage(c + 1, 1)
            drain(0)
            writeback(c, 0)
            stage(c + 2, 0)
            drain(1)
            writeback(c + 1, 1)
            return carry

        lax.fori_loop(0, (n_chunks - 1) // 2, outer, 0)
        if n_chunks % 2 == 1:
            drain(0)
            writeback(n_chunks - 1, 0)
        else:
            stage(n_chunks - 1, 1)
            drain(0)
            writeback(n_chunks - 2, 0)
            drain(1)
            writeback(n_chunks - 1, 1)

    return gather_kernel(c0, c1, c2, leaf_table, path_table)


def _segment_fused(p0, p1, mode, idx2, bounds, w1t, w2t, w3t, a2, w_out,
                   b_out2, num_blocks, n_total):
    p0 = tuple(p0)
    p1 = tuple(p1) if p1 is not None else None
    ll0, pm0, lr0 = p0
    two = p1 is not None
    d = ll0.shape[1]
    code = w1t.shape[1]
    out_dim = w_out.shape[0]
    seg_pad = num_blocks * SEG_BLOCK
    C = ROW_CHUNK
    S = SPLIT_ROWS
    c_split = S // C
    assert c_split * C == S and n_total % NUM_WORKERS == 0

    def body(*refs):
        nin = 9 + (3 if two else 0)
        (bounds_r, *data_r) = refs[:1 + (6 if two else 3)]
        i_r, w1_r, w2_r, w3_r, a_r, wout_r, bout_r = refs[
            1 + (6 if two else 3):nin + 2]
        out_r = refs[nin + 2]
        lbuf, pbuf, rbuf, ibuf, sem_l, sem_p, sem_r, sem_i = refs[nin + 3:]
        srcs0 = tuple(data_r[:3])
        srcs1 = tuple(data_r[3:6]) if two else None

        b = pl.program_id(0)
        r0 = bounds_r[b]
        r1 = bounds_r[b + 1]
        if mode == "all":
            process = None
            c_lo = r0 // C
            c_hi = lax.div(r1 + C - 1, C)
        else:
            process = (r1 <= S) if mode == "lower" else (r1 > S)
            c_lo = jnp.where(process, r0 // C, 0)
            c_hi = jnp.where(process, lax.div(r1 + C - 1, C), 0)
        seg0 = b * SEG_BLOCK
        bufs = (lbuf, pbuf, rbuf)
        data_sems = (sem_l, sem_p, sem_r)

        def glob_off(c):
            return jnp.minimum(c * C, n_total - C)

        def copies(c, slot, act):
            off = glob_off(c)
            if two:
                @pl.when(c < c_split)
                def _():
                    for t in range(3):
                        act(pltpu.make_async_copy(
                            srcs0[t].at[pl.ds(off, C)], bufs[t].at[slot],
                            data_sems[t].at[slot]))

                @pl.when(c >= c_split)
                def _():
                    for t in range(3):
                        act(pltpu.make_async_copy(
                            srcs1[t].at[pl.ds(off - S, C)], bufs[t].at[slot],
                            data_sems[t].at[slot]))
            else:
                for t in range(3):
                    act(pltpu.make_async_copy(
                        srcs0[t].at[pl.ds(off, C)], bufs[t].at[slot],
                        data_sems[t].at[slot]))
            act(pltpu.make_async_copy(
                i_r.at[:, pl.ds(off, C)], ibuf.at[slot], sem_i.at[slot]))

        @pl.when(c_lo < c_hi)
        def _():
            copies(c_lo, 0, lambda cp: cp.start())

        def chunk(c, carry):
            m, dnm, acc = carry
            slot = lax.rem(c - c_lo, 2)

            @pl.when(c + 1 < c_hi)
            def _():
                copies(c + 1, 1 - slot, lambda cp: cp.start())

            copies(c, slot, lambda cp: cp.wait())
            z = jnp.dot(lbuf[slot], w1_r[...],
                        preferred_element_type=jnp.float32)
            z = z + jnp.dot(pbuf[slot], w2_r[...],
                            preferred_element_type=jnp.float32)
            z = z + jnp.dot(rbuf[slot], w3_r[...],
                            preferred_element_type=jnp.float32)
            h = jnp.tanh(z)
            sc = lax.dot_general(a_r[...], h, (((1,), (1,)), ((), ())),
                                 preferred_element_type=jnp.float32)
            pos = glob_off(c) + lax.broadcasted_iota(jnp.int32, (1, C), 1)
            rel = jnp.where(pos >= c * C, ibuf[slot] - seg0, -1)
            rows = lax.broadcasted_iota(jnp.int32, (SEG_BLOCK, C), 0)
            onehot = rel == rows
            mc = jnp.max(jnp.where(onehot, sc, -1e30), axis=1, keepdims=True)
            m_new = jnp.maximum(m, mc)
            alpha = jnp.exp(m - m_new)
            ex = jnp.exp(jnp.where(onehot, sc - m_new, -1e30))
            dnm = dnm * alpha + jnp.sum(ex, axis=1, keepdims=True)
            acc = acc * alpha + jnp.dot(
                ex, h, preferred_element_type=jnp.float32)
            return m_new, dnm, acc

        m0 = jnp.full((SEG_BLOCK, 1), -1e30, jnp.float32)
        d0 = jnp.zeros((SEG_BLOCK, 1), jnp.float32)
        a0 = jnp.zeros((SEG_BLOCK, code), jnp.float32)
        m, dnm, acc = lax.fori_loop(c_lo, c_hi, chunk, (m0, d0, a0))
        v = jnp.where(dnm > 0, acc / jnp.where(dnm > 0, dnm, 1.0), 0.0)
        out = lax.dot_general(
            v, wout_r[...], (((1,), (1,)), ((), ())),
            preferred_element_type=jnp.float32)
        out = out + bout_r[...]
        if process is not None:
            out = jnp.where(process, out, 0.0)
        out_r[...] = out

    hbm = pl.BlockSpec(memory_space=pltpu.MemorySpace.HBM)
    in_specs = ([pl.BlockSpec(memory_space=pltpu.MemorySpace.SMEM)]
                + [hbm] * (6 if two else 3)
                + [hbm,
                   pl.BlockSpec((d, code), lambda b: (0, 0)),
                   pl.BlockSpec((d, code), lambda b: (0, 0)),
                   pl.BlockSpec((d, code), lambda b: (0, 0)),
                   pl.BlockSpec((1, code), lambda b: (0, 0)),
                   pl.BlockSpec((out_dim, code), lambda b: (0, 0)),
                   pl.BlockSpec((1, out_dim), lambda b: (0, 0))])
    args = ((bounds,) + p0 + (p1 if two else ())
            + (idx2, w1t, w2t, w3t, a2, w_out, b_out2))
    return pl.pallas_call(
        body,
        grid=(num_blocks,),
        in_specs=in_specs,
        out_specs=pl.BlockSpec((SEG_BLOCK, out_dim), lambda b: (b, 0)),
        out_shape=jax.ShapeDtypeStruct((seg_pad, out_dim), jnp.float32),
        scratch_shapes=[
            pltpu.VMEM((2, C, d), ll0.dtype),
            pltpu.VMEM((2, C, d), ll0.dtype),
            pltpu.VMEM((2, C, d), ll0.dtype),
            pltpu.VMEM((2, 1, C), jnp.int32),
            pltpu.SemaphoreType.DMA((2,)),
            pltpu.SemaphoreType.DMA((2,)),
            pltpu.SemaphoreType.DMA((2,)),
            pltpu.SemaphoreType.DMA((2,)),
        ],
    )(*args)


def kernel(contexts, indices, leaf_table, path_table, W_fc, a, W_out, b_out):
    n = contexts.shape[0]
    d = leaf_table.shape[1]
    num_blocks = (NUM_SEG + SEG_BLOCK - 1) // SEG_BLOCK
    s = SPLIT_ROWS

    c0 = contexts[:, 0]
    c1 = contexts[:, 1]
    c2 = contexts[:, 2]
    pa = _sc_gather(c0[:s], c1[:s], c2[:s], leaf_table, path_table)
    pb = _sc_gather(c0[s:], c1[s:], c2[s:], leaf_table, path_table)

    wt = W_fc.T
    seg_starts = jnp.arange(num_blocks, dtype=jnp.int32) * SEG_BLOCK
    bounds = jnp.concatenate([
        jnp.searchsorted(indices, seg_starts).astype(jnp.int32),
        jnp.array([n], jnp.int32),
    ])
    idx2 = indices[None, :]
    out_lo = _segment_fused(pa, None, "lower", idx2, bounds,
                            wt[:d], wt[d:2 * d], wt[2 * d:], a[None, :],
                            W_out, b_out[None, :], num_blocks, n)
    out_hi = _segment_fused(pa, pb, "upper", idx2, bounds,
                            wt[:d], wt[d:2 * d], wt[2 * d:], a[None, :],
                            W_out, b_out[None, :], num_blocks, n)
    return (out_lo + out_hi)[:NUM_SEG]


# speedup vs baseline: 1.1728x; 1.0012x over previous
import functools

import jax
import jax.numpy as jnp
from jax import lax
from jax.experimental import pallas as pl
from jax.experimental.pallas import tpu as pltpu
from jax.experimental.pallas import tpu_sc as plsc

NUM_SEG = 10000
SEG_BLOCK = 256
ROW_CHUNK = 1024
SC_CHUNK = 80
NUM_WORKERS = 32
SPLIT_ROWS = 158720


def _sc_gather(c0, c1, c2, leaf_table, path_table):
    n = c0.shape[0]
    d = leaf_table.shape[1]
    per_w = n // NUM_WORKERS
    B = SC_CHUNK
    n_chunks = per_w // B
    assert n_chunks >= 3 and per_w % B == 0 and n % NUM_WORKERS == 0
    mesh = plsc.VectorSubcoreMesh(core_axis_name="c", subcore_axis_name="s")
    row_t = jax.ShapeDtypeStruct((n, d), leaf_table.dtype)
    scr = ([pltpu.VMEM((B,), jnp.int32)] * 6
           + [pltpu.VMEM((B, d), leaf_table.dtype)] * 6
           + [pltpu.SemaphoreType.DMA] * 6)

    @functools.partial(pl.kernel, mesh=mesh, out_type=(row_t, row_t, row_t),
                       scratch_types=scr)
    def gather_kernel(c0_h, c1_h, c2_h, leaf_h, path_h, o0_h, o1_h, o2_h,
                      i00, i01, i02, i10, i11, i12,
                      r00, r01, r02, r10, r11, r12,
                      s00, s01, s02, s10, s11, s12):
        ibufs = ((i00, i01, i02), (i10, i11, i12))
        rbufs = ((r00, r01, r02), (r10, r11, r12))
        sems = ((s00, s01, s02), (s10, s11, s12))
        idxs = (c0_h, c1_h, c2_h)
        tabs = (leaf_h, path_h, leaf_h)
        outs = (o0_h, o1_h, o2_h)
        wid = lax.axis_index("s") * 2 + lax.axis_index("c")
        base = wid * per_w

        def stage(c, slot):
            off = base + c * B
            for t in range(3):
                pltpu.sync_copy(idxs[t].at[pl.ds(off, B)], ibufs[slot][t])
            for t in range(3):
                pltpu.async_copy(tabs[t].at[ibufs[slot][t]], rbufs[slot][t],
                                 sems[slot][t])

        def drain(slot):
            for t in range(3):
                pltpu.make_async_copy(tabs[t].at[ibufs[slot][t]],
                                      rbufs[slot][t], sems[slot][t]).wait()

        def writeback(c, slot):
            off = base + c * B
            for t in range(3):
                pltpu.sync_copy(rbufs[slot][t], outs[t].at[pl.ds(off, B)])

        stage(0, 0)

        def outer(g, carry):
            c = 2 * g
            stage(c + 1, 1)
            drain(0)
            writeback(c, 0)
            stage(c + 2, 0)
            drain(1)
            writeback(c + 1, 1)
            return carry

        lax.fori_loop(0, (n_chunks - 1) // 2, outer, 0)
        if n_chunks % 2 == 1:
            drain(0)
            writeback(n_chunks - 1, 0)
        else:
            stage(n_chunks - 1, 1)
            drain(0)
            writeback(n_chunks - 2, 0)
            drain(1)
            writeback(n_chunks - 1, 1)

    return gather_kernel(c0, c1, c2, leaf_table, path_table)


def _segment_fused(p0, p1, mode, idx2, bounds, w1t, w2t, w3t, a2, w_out,
                   b_out2, num_blocks, n_total):
    p0 = tuple(p0)
    p1 = tuple(p1) if p1 is not None else None
    ll0, pm0, lr0 = p0
    two = p1 is not None
    d = ll0.shape[1]
    code = w1t.shape[1]
    out_dim = w_out.shape[0]
    seg_pad = num_blocks * SEG_BLOCK
    C = ROW_CHUNK
    S = SPLIT_ROWS
    c_split = S // C
    assert c_split * C == S and n_total % NUM_WORKERS == 0

    def body(*refs):
        nin = 9 + (3 if two else 0)
        (bounds_r, *data_r) = refs[:1 + (6 if two else 3)]
        i_r, w1_r, w2_r, w3_r, a_r, wout_r, bout_r = refs[
            1 + (6 if two else 3):nin + 2]
        out_r = refs[nin + 2]
        lbuf, pbuf, rbuf, ibuf, sem_l, sem_p, sem_r, sem_i = refs[nin + 3:]
        srcs0 = tuple(data_r[:3])
        srcs1 = tuple(data_r[3:6]) if two else None

        b = pl.program_id(0)
        r0 = bounds_r[b]
        r1 = bounds_r[b + 1]
        if mode == "all":
            process = None
            c_lo = r0 // C
            c_hi = lax.div(r1 + C - 1, C)
        else:
            process = (r1 <= S) if mode == "lower" else (r1 > S)
            c_lo = jnp.where(process, r0 // C, 0)
            c_hi = jnp.where(process, lax.div(r1 + C - 1, C), 0)
        seg0 = b * SEG_BLOCK
        bufs = (lbuf, pbuf, rbuf)
        data_sems = (sem_l, sem_p, sem_r)

        def glob_off(c):
            return jnp.minimum(c * C, n_total - C)

        def copies(c, slot, act):
            off = glob_off(c)
            if two:
                @pl.when(c < c_split)
                def _():
                    for t in range(3):
                        act(pltpu.make_async_copy(
                            srcs0[t].at[pl.ds(off, C)], bufs[t].at[slot],
                            data_sems[t].at[slot]))

                @pl.when(c >= c_split)
                def _():
                    for t in range(3):
                        act(pltpu.make_async_copy(
                            srcs1[t].at[pl.ds(off - S, C)], bufs[t].at[slot],
                            data_sems[t].at[slot]))
            else:
                for t in range(3):
                    act(pltpu.make_async_copy(
                        srcs0[t].at[pl.ds(off, C)], bufs[t].at[slot],
                        data_sems[t].at[slot]))
            act(pltpu.make_async_copy(
                i_r.at[:, pl.ds(off, C)], ibuf.at[slot], sem_i.at[slot]))

        @pl.when(c_lo < c_hi)
        def _():
            copies(c_lo, 0, lambda cp: cp.start())

        def chunk(c, carry):
            m, dnm, acc = carry
            slot = lax.rem(c - c_lo, 2)

            @pl.when(c + 1 < c_hi)
            def _():
                copies(c + 1, 1 - slot, lambda cp: cp.start())

            copies(c, slot, lambda cp: cp.wait())
            z = jnp.dot(lbuf[slot], w1_r[...],
                        preferred_element_type=jnp.float32)
            z = z + jnp.dot(pbuf[slot], w2_r[...],
                            preferred_element_type=jnp.float32)
            z = z + jnp.dot(rbuf[slot], w3_r[...],
                            preferred_element_type=jnp.float32)
            h = jnp.tanh(z)
            sc = lax.dot_general(a_r[...], h, (((1,), (1,)), ((), ())),
                                 preferred_element_type=jnp.float32)
            pos = glob_off(c) + lax.broadcasted_iota(jnp.int32, (1, C), 1)
            rel = jnp.where(pos >= c * C, ibuf[slot] - seg0, -1)
            rows = lax.broadcasted_iota(jnp.int32, (SEG_BLOCK, C), 0)
            onehot = rel == rows
            mc = jnp.max(jnp.where(onehot, sc, -1e30), axis=1, keepdims=True)
            m_new = jnp.maximum(m, mc)
            alpha = jnp.exp(m - m_new)
            ex = jnp.exp(jnp.where(onehot, sc - m_new, -1e30))
            dnm = dnm * alpha + jnp.sum(ex, axis=1, keepdims=True)
            acc = acc * alpha + jnp.dot(
                ex, h, preferred_element_type=jnp.float32)
            return m_new, dnm, acc

        m0 = jnp.full((SEG_BLOCK, 1), -1e30, jnp.float32)
        d0 = jnp.zeros((SEG_BLOCK, 1), jnp.float32)
        a0 = jnp.zeros((SEG_BLOCK, code), jnp.float32)
        m, dnm, acc = lax.fori_loop(c_lo, c_hi, chunk, (m0, d0, a0))
        v = jnp.where(dnm > 0, acc / jnp.where(dnm > 0, dnm, 1.0), 0.0)
        out = lax.dot_general(
            v, wout_r[...], (((1,), (1,)), ((), ())),
            preferred_element_type=jnp.float32)
        out = out + bout_r[...]
        if process is not None:
            out = jnp.where(process, out, 0.0)
        out_r[...] = out

    hbm = pl.BlockSpec(memory_space=pltpu.MemorySpace.HBM)
    in_specs = ([pl.BlockSpec(memory_space=pltpu.MemorySpace.SMEM)]
                + [hbm] * (6 if two else 3)
                + [hbm,
                   pl.BlockSpec((d, code), lambda b: (0, 0)),
                   pl.BlockSpec((d, code), lambda b: (0, 0)),
                   pl.BlockSpec((d, code), lambda b: (0, 0)),
                   pl.BlockSpec((1, code), lambda b: (0, 0)),
                   pl.BlockSpec((out_dim, code), lambda b: (0, 0)),
                   pl.BlockSpec((1, out_dim), lambda b: (0, 0))])
    args = ((bounds,) + p0 + (p1 if two else ())
            + (idx2, w1t, w2t, w3t, a2, w_out, b_out2))
    return pl.pallas_call(
        body,
        grid=(num_blocks,),
        in_specs=in_specs,
        out_specs=pl.BlockSpec((SEG_BLOCK, out_dim), lambda b: (b, 0)),
        out_shape=jax.ShapeDtypeStruct((seg_pad, out_dim), jnp.float32),
        scratch_shapes=[
            pltpu.VMEM((2, C, d), ll0.dtype),
            pltpu.VMEM((2, C, d), ll0.dtype),
            pltpu.VMEM((2, C, d), ll0.dtype),
            pltpu.VMEM((2, 1, C), jnp.int32),
            pltpu.SemaphoreType.DMA((2,)),
            pltpu.SemaphoreType.DMA((2,)),
            pltpu.SemaphoreType.DMA((2,)),
            pltpu.SemaphoreType.DMA((2,)),
        ],
    )(*args)


def kernel(contexts, indices, leaf_table, path_table, W_fc, a, W_out, b_out):
    n = contexts.shape[0]
    d = leaf_table.shape[1]
    num_blocks = (NUM_SEG + SEG_BLOCK - 1) // SEG_BLOCK
    s = SPLIT_ROWS

    ctx_t = contexts.T
    c0 = ctx_t[0]
    c1 = ctx_t[1]
    c2 = ctx_t[2]
    pa = _sc_gather(c0[:s], c1[:s], c2[:s], leaf_table, path_table)
    pb = _sc_gather(c0[s:], c1[s:], c2[s:], leaf_table, path_table)

    wt = W_fc.T
    seg_starts = jnp.arange(num_blocks, dtype=jnp.int32) * SEG_BLOCK
    bounds = jnp.concatenate([
        jnp.searchsorted(indices, seg_starts).astype(jnp.int32),
        jnp.array([n], jnp.int32),
    ])
    idx2 = indices[None, :]
    out_lo = _segment_fused(pa, None, "lower", idx2, bounds,
                            wt[:d], wt[d:2 * d], wt[2 * d:], a[None, :],
                            W_out, b_out[None, :], num_blocks, n)
    out_hi = _segment_fused(pa, pb, "upper", idx2, bounds,
                            wt[:d], wt[d:2 * d], wt[2 * d:], a[None, :],
                            W_out, b_out[None, :], num_blocks, n)
    return (out_lo + out_hi)[:NUM_SEG]


# speedup vs baseline: 1.1859x; 1.0111x over previous
import functools

import jax
import jax.numpy as jnp
from jax import lax
from jax.experimental import pallas as pl
from jax.experimental.pallas import tpu as pltpu
from jax.experimental.pallas import tpu_sc as plsc

NUM_SEG = 10000
SEG_BLOCK = 256
ROW_CHUNK = 2048
SC_CHUNK = 80
NUM_WORKERS = 32
SPLIT_ROWS = 153600


def _sc_gather(c0, c1, c2, leaf_table, path_table):
    n = c0.shape[0]
    d = leaf_table.shape[1]
    per_w = n // NUM_WORKERS
    B = SC_CHUNK
    n_chunks = per_w // B
    assert n_chunks >= 3 and per_w % B == 0 and n % NUM_WORKERS == 0
    mesh = plsc.VectorSubcoreMesh(core_axis_name="c", subcore_axis_name="s")
    row_t = jax.ShapeDtypeStruct((n, d), leaf_table.dtype)
    scr = ([pltpu.VMEM((B,), jnp.int32)] * 6
           + [pltpu.VMEM((B, d), leaf_table.dtype)] * 6
           + [pltpu.SemaphoreType.DMA] * 6)

    @functools.partial(pl.kernel, mesh=mesh, out_type=(row_t, row_t, row_t),
                       scratch_types=scr)
    def gather_kernel(c0_h, c1_h, c2_h, leaf_h, path_h, o0_h, o1_h, o2_h,
                      i00, i01, i02, i10, i11, i12,
                      r00, r01, r02, r10, r11, r12,
                      s00, s01, s02, s10, s11, s12):
        ibufs = ((i00, i01, i02), (i10, i11, i12))
        rbufs = ((r00, r01, r02), (r10, r11, r12))
        sems = ((s00, s01, s02), (s10, s11, s12))
        idxs = (c0_h, c1_h, c2_h)
        tabs = (leaf_h, path_h, leaf_h)
        outs = (o0_h, o1_h, o2_h)
        wid = lax.axis_index("s") * 2 + lax.axis_index("c")
        base = wid * per_w

        def stage(c, slot):
            off = base + c * B
            for t in range(3):
                pltpu.sync_copy(idxs[t].at[pl.ds(off, B)], ibufs[slot][t])
            for t in range(3):
                pltpu.async_copy(tabs[t].at[ibufs[slot][t]], rbufs[slot][t],
                                 sems[slot][t])

        def drain(slot):
            for t in range(3):
                pltpu.make_async_copy(tabs[t].at[ibufs[slot][t]],
                                      rbufs[slot][t], sems[slot][t]).wait()

        def writeback(c, slot):
            off = base + c * B
            for t in range(3):
                pltpu.sync_copy(rbufs[slot][t], outs[t].at[pl.ds(off, B)])

        stage(0, 0)

        def outer(g, carry):
            c = 2 * g
            stage(c + 1, 1)
            drain(0)
            writeback(c, 0)
            stage(c + 2, 0)
            drain(1)
            writeback(c + 1, 1)
            return carry

        lax.fori_loop(0, (n_chunks - 1) // 2, outer, 0)
        if n_chunks % 2 == 1:
            drain(0)
            writeback(n_chunks - 1, 0)
        else:
            stage(n_chunks - 1, 1)
            drain(0)
            writeback(n_chunks - 2, 0)
            drain(1)
            writeback(n_chunks - 1, 1)

    return gather_kernel(c0, c1, c2, leaf_table, path_table)


def _segment_fused(p0, p1, mode, idx2, bounds, w1t, w2t, w3t, a2, w_out,
                   b_out2, num_blocks, n_total):
    p0 = tuple(p0)
    p1 = tuple(p1) if p1 is not None else None
    ll0, pm0, lr0 = p0
    two = p1 is not None
    d = ll0.shape[1]
    code = w1t.shape[1]
    out_dim = w_out.shape[0]
    seg_pad = num_blocks * SEG_BLOCK
    C = ROW_CHUNK
    S = SPLIT_ROWS
    c_split = S // C
    assert c_split * C == S and n_total % NUM_WORKERS == 0

    def body(*refs):
        nin = 9 + (3 if two else 0)
        (bounds_r, *data_r) = refs[:1 + (6 if two else 3)]
        i_r, w1_r, w2_r, w3_r, a_r, wout_r, bout_r = refs[
            1 + (6 if two else 3):nin + 2]
        out_r = refs[nin + 2]
        lbuf, pbuf, rbuf, ibuf, sem_l, sem_p, sem_r, sem_i = refs[nin + 3:]
        srcs0 = tuple(data_r[:3])
        srcs1 = tuple(data_r[3:6]) if two else None

        b = pl.program_id(0)
        r0 = bounds_r[b]
        r1 = bounds_r[b + 1]
        if mode == "all":
            process = None
            c_lo = r0 // C
            c_hi = lax.div(r1 + C - 1, C)
        else:
            process = (r1 <= S) if mode == "lower" else (r1 > S)
            c_lo = jnp.where(process, r0 // C, 0)
            c_hi = jnp.where(process, lax.div(r1 + C - 1, C), 0)
        seg0 = b * SEG_BLOCK
        bufs = (lbuf, pbuf, rbuf)
        data_sems = (sem_l, sem_p, sem_r)

        def glob_off(c):
            return jnp.minimum(c * C, n_total - C)

        def copies(c, slot, act):
            off = glob_off(c)
            if two:
                @pl.when(c < c_split)
                def _():
                    for t in range(3):
                        act(pltpu.make_async_copy(
                            srcs0[t].at[pl.ds(off, C)], bufs[t].at[slot],
                            data_sems[t].at[slot]))

                @pl.when(c >= c_split)
                def _():
                    for t in range(3):
                        act(pltpu.make_async_copy(
                            srcs1[t].at[pl.ds(off - S, C)], bufs[t].at[slot],
                            data_sems[t].at[slot]))
            else:
                for t in range(3):
                    act(pltpu.make_async_copy(
                        srcs0[t].at[pl.ds(off, C)], bufs[t].at[slot],
                        data_sems[t].at[slot]))
            act(pltpu.make_async_copy(
                i_r.at[:, pl.ds(off, C)], ibuf.at[slot], sem_i.at[slot]))

        @pl.when(c_lo < c_hi)
        def _():
            copies(c_lo, 0, lambda cp: cp.start())

        def chunk(c, carry):
            m, dnm, acc = carry
            slot = lax.rem(c - c_lo, 2)

            @pl.when(c + 1 < c_hi)
            def _():
                copies(c + 1, 1 - slot, lambda cp: cp.start())

            copies(c, slot, lambda cp: cp.wait())
            z = jnp.dot(lbuf[slot], w1_r[...],
                        preferred_element_type=jnp.float32)
            z = z + jnp.dot(pbuf[slot], w2_r[...],
                            preferred_element_type=jnp.float32)
            z = z + jnp.dot(rbuf[slot], w3_r[...],
                            preferred_element_type=jnp.float32)
            h = jnp.tanh(z)
            sc = lax.dot_general(a_r[...], h, (((1,), (1,)), ((), ())),
                                 preferred_element_type=jnp.float32)
            pos = glob_off(c) + lax.broadcasted_iota(jnp.int32, (1, C), 1)
            rel = jnp.where(pos >= c * C, ibuf[slot] - seg0, -1)
            rows = lax.broadcasted_iota(jnp.int32, (SEG_BLOCK, C), 0)
            onehot = rel == rows
            mc = jnp.max(jnp.where(onehot, sc, -1e30), axis=1, keepdims=True)
            m_new = jnp.maximum(m, mc)
            alpha = jnp.exp(m - m_new)
            ex = jnp.exp(jnp.where(onehot, sc - m_new, -1e30))
            dnm = dnm * alpha + jnp.sum(ex, axis=1, keepdims=True)
            acc = acc * alpha + jnp.dot(
                ex, h, preferred_element_type=jnp.float32)
            return m_new, dnm, acc

        m0 = jnp.full((SEG_BLOCK, 1), -1e30, jnp.float32)
        d0 = jnp.zeros((SEG_BLOCK, 1), jnp.float32)
        a0 = jnp.zeros((SEG_BLOCK, code), jnp.float32)
        m, dnm, acc = lax.fori_loop(c_lo, c_hi, chunk, (m0, d0, a0))
        v = jnp.where(dnm > 0, acc / jnp.where(dnm > 0, dnm, 1.0), 0.0)
        out = lax.dot_general(
            v, wout_r[...], (((1,), (1,)), ((), ())),
            preferred_element_type=jnp.float32)
        out = out + bout_r[...]
        if process is not None:
            out = jnp.where(process, out, 0.0)
        out_r[...] = out

    hbm = pl.BlockSpec(memory_space=pltpu.MemorySpace.HBM)
    in_specs = ([pl.BlockSpec(memory_space=pltpu.MemorySpace.SMEM)]
                + [hbm] * (6 if two else 3)
                + [hbm,
                   pl.BlockSpec((d, code), lambda b: (0, 0)),
                   pl.BlockSpec((d, code), lambda b: (0, 0)),
                   pl.BlockSpec((d, code), lambda b: (0, 0)),
                   pl.BlockSpec((1, code), lambda b: (0, 0)),
                   pl.BlockSpec((out_dim, code), lambda b: (0, 0)),
                   pl.BlockSpec((1, out_dim), lambda b: (0, 0))])
    args = ((bounds,) + p0 + (p1 if two else ())
            + (idx2, w1t, w2t, w3t, a2, w_out, b_out2))
    return pl.pallas_call(
        body,
        grid=(num_blocks,),
        in_specs=in_specs,
        out_specs=pl.BlockSpec((SEG_BLOCK, out_dim), lambda b: (b, 0)),
        out_shape=jax.ShapeDtypeStruct((seg_pad, out_dim), jnp.float32),
        scratch_shapes=[
            pltpu.VMEM((2, C, d), ll0.dtype),
            pltpu.VMEM((2, C, d), ll0.dtype),
            pltpu.VMEM((2, C, d), ll0.dtype),
            pltpu.VMEM((2, 1, C), jnp.int32),
            pltpu.SemaphoreType.DMA((2,)),
            pltpu.SemaphoreType.DMA((2,)),
            pltpu.SemaphoreType.DMA((2,)),
            pltpu.SemaphoreType.DMA((2,)),
        ],
    )(*args)


def kernel(contexts, indices, leaf_table, path_table, W_fc, a, W_out, b_out):
    n = contexts.shape[0]
    d = leaf_table.shape[1]
    num_blocks = (NUM_SEG + SEG_BLOCK - 1) // SEG_BLOCK
    s = SPLIT_ROWS

    ctx_t = contexts.T
    c0 = ctx_t[0]
    c1 = ctx_t[1]
    c2 = ctx_t[2]
    pa = _sc_gather(c0[:s], c1[:s], c2[:s], leaf_table, path_table)
    pb = _sc_gather(c0[s:], c1[s:], c2[s:], leaf_table, path_table)

    wt = W_fc.T
    seg_starts = jnp.arange(num_blocks, dtype=jnp.int32) * SEG_BLOCK
    bounds = jnp.concatenate([
        jnp.searchsorted(indices, seg_starts).astype(jnp.int32),
        jnp.array([n], jnp.int32),
    ])
    idx2 = indices[None, :]
    out_lo = _segment_fused(pa, None, "lower", idx2, bounds,
                            wt[:d], wt[d:2 * d], wt[2 * d:], a[None, :],
                            W_out, b_out[None, :], num_blocks, n)
    out_hi = _segment_fused(pa, pb, "upper", idx2, bounds,
                            wt[:d], wt[d:2 * d], wt[2 * d:], a[None, :],
                            W_out, b_out[None, :], num_blocks, n)
    return (out_lo + out_hi)[:NUM_SEG]
